# baseline (speedup 1.0000x reference)
import functools

import jax
import jax.numpy as jnp
from jax import lax
from jax.experimental import pallas as pl
from jax.experimental.pallas import tpu as pltpu
from jax.experimental.pallas import tpu_sc as plsc

B = 320000
D = 128
NW = 32
E_PER_W = B // NW
C = 400
N_CHUNKS = E_PER_W // C
GROUPS = C // 16

_mesh = plsc.VectorSubcoreMesh(core_axis_name="c", subcore_axis_name="s")


@functools.partial(
    pl.kernel,
    out_type=jax.ShapeDtypeStruct((B,), jnp.float32),
    mesh=_mesh,
    scratch_types=[
        pltpu.VMEM((C,), jnp.int32),
        pltpu.VMEM((C,), jnp.int32),
        pltpu.VMEM((C, D), jnp.float32),
        pltpu.VMEM((C, D), jnp.float32),
        pltpu.VMEM((C,), jnp.float32),
        pltpu.SemaphoreType.DMA,
    ],
    compiler_params=pltpu.CompilerParams(needs_layout_passes=False),
)
def _sc_kernel(x_sotu_hbm, x_taxon_hbm, idx0_hbm, idx1_hbm, out_hbm,
               idx0_v, idx1_v, rows0_v, rows1_v, out_v, sem):
    wid = lax.axis_index("s") * 2 + lax.axis_index("c")
    lane = lax.iota(jnp.int32, 16)

    def chunk_body(it, _):
        base = wid * E_PER_W + it * C
        pltpu.sync_copy(idx0_hbm.at[pl.ds(base, C)], idx0_v)
        pltpu.sync_copy(idx1_hbm.at[pl.ds(base, C)], idx1_v)
        cp0 = pltpu.async_copy(x_sotu_hbm.at[idx0_v], rows0_v, sem)
        cp1 = pltpu.async_copy(x_taxon_hbm.at[idx1_v], rows1_v, sem)
        cp0.wait()
        cp1.wait()

        def group_body(g, _):
            row_idx = lane + g * 16
            eight = jnp.full((16,), 8, jnp.int32)
            cols = [jnp.full((16,), k, jnp.int32) for k in range(8)]
            accs = [jnp.zeros((16,), jnp.float32) for _ in range(8)]
            for d in range(D):
                k = d & 7
                a = plsc.load_gather(rows0_v, [row_idx, cols[k]])
                b = plsc.load_gather(rows1_v, [row_idx, cols[k]])
                accs[k] = accs[k] + a * b
                cols[k] = cols[k] + eight
            s01 = accs[0] + accs[1]
            s23 = accs[2] + accs[3]
            s45 = accs[4] + accs[5]
            s67 = accs[6] + accs[7]
            out_v[pl.ds(g * 16, 16)] = (s01 + s23) + (s45 + s67)
            return 0

        lax.fori_loop(0, GROUPS, group_body, 0)
        pltpu.sync_copy(out_v, out_hbm.at[pl.ds(base, C)])
        return 0

    lax.fori_loop(0, N_CHUNKS, chunk_body, 0)


def kernel(x_sotu, x_taxon, edge_label_index):
    idx0 = edge_label_index[0]
    idx1 = edge_label_index[1]
    return _sc_kernel(x_sotu, x_taxon, idx0, idx1)


# speedup vs baseline: 3.8890x; 3.8890x over previous
import functools

import jax
import jax.numpy as jnp
from jax import lax
from jax.experimental import pallas as pl
from jax.experimental.pallas import tpu as pltpu
from jax.experimental.pallas import tpu_sc as plsc

B = 320000
D = 128
NW = 32
E_PER_W = B // NW
C = 400
N_CHUNKS = E_PER_W // C
GROUPS = C // 16

_mesh = plsc.VectorSubcoreMesh(core_axis_name="c", subcore_axis_name="s")


@functools.partial(
    pl.kernel,
    out_type=jax.ShapeDtypeStruct((B,), jnp.float32),
    mesh=_mesh,
    scratch_types=[
        pltpu.VMEM((C,), jnp.int32),
        pltpu.VMEM((C,), jnp.int32),
        pltpu.VMEM((C, D), jnp.float32),
        pltpu.VMEM((C, D), jnp.float32),
        pltpu.VMEM((C,), jnp.float32),
        pltpu.VMEM((16, 17), jnp.float32),
        pltpu.SemaphoreType.DMA,
    ],
    compiler_params=pltpu.CompilerParams(needs_layout_passes=False),
)
def _sc_kernel(x_sotu_hbm, x_taxon_hbm, idx0_hbm, idx1_hbm, out_hbm,
               idx0_v, idx1_v, rows0_v, rows1_v, out_v, tr_v, sem):
    wid = lax.axis_index("s") * 2 + lax.axis_index("c")
    lane = lax.iota(jnp.int32, 16)

    def chunk_body(it, _):
        base = wid * E_PER_W + it * C
        pltpu.sync_copy(idx0_hbm.at[pl.ds(base, C)], idx0_v)
        pltpu.sync_copy(idx1_hbm.at[pl.ds(base, C)], idx1_v)
        cp0 = pltpu.async_copy(x_sotu_hbm.at[idx0_v], rows0_v, sem)
        cp1 = pltpu.async_copy(x_taxon_hbm.at[idx1_v], rows1_v, sem)
        cp0.wait()
        cp1.wait()

        def group_body(g, _):
            for e in range(16):
                row = g * 16 + e
                ps = []
                for k in range(8):
                    a = rows0_v[row, pl.ds(16 * k, 16)]
                    b = rows1_v[row, pl.ds(16 * k, 16)]
                    ps.append(a * b)
                s01 = ps[0] + ps[1]
                s23 = ps[2] + ps[3]
                s45 = ps[4] + ps[5]
                s67 = ps[6] + ps[7]
                tr_v[e, pl.ds(0, 16)] = (s01 + s23) + (s45 + s67)
            one = jnp.ones((16,), jnp.int32)
            col = jnp.zeros((16,), jnp.int32)
            acc = jnp.zeros((16,), jnp.float32)
            for c in range(16):
                acc = acc + plsc.load_gather(tr_v, [lane, col])
                col = col + one
            out_v[pl.ds(g * 16, 16)] = acc
            return 0

        lax.fori_loop(0, GROUPS, group_body, 0)
        pltpu.sync_copy(out_v, out_hbm.at[pl.ds(base, C)])
        return 0

    lax.fori_loop(0, N_CHUNKS, chunk_body, 0)


def kernel(x_sotu, x_taxon, edge_label_index):
    idx0 = edge_label_index[0]
    idx1 = edge_label_index[1]
    return _sc_kernel(x_sotu, x_taxon, idx0, idx1)


# speedup vs baseline: 5.9194x; 1.5221x over previous
import functools

import jax
import jax.numpy as jnp
from jax import lax
from jax.experimental import pallas as pl
from jax.experimental.pallas import tpu as pltpu
from jax.experimental.pallas import tpu_sc as plsc

B = 320000
D = 128
NW = 32
E_PER_W = B // NW
C = 80
N_CHUNKS = E_PER_W // C
GROUPS = C // 16

_mesh = plsc.VectorSubcoreMesh(core_axis_name="c", subcore_axis_name="s")


@functools.partial(
    pl.kernel,
    out_type=jax.ShapeDtypeStruct((B,), jnp.float32),
    mesh=_mesh,
    scratch_types=[
        pltpu.VMEM((E_PER_W,), jnp.int32),
        pltpu.VMEM((E_PER_W,), jnp.int32),
        pltpu.VMEM((E_PER_W,), jnp.float32),
        pltpu.VMEM((C, D), jnp.float32),
        pltpu.VMEM((C, D), jnp.float32),
        pltpu.VMEM((C, D), jnp.float32),
        pltpu.VMEM((C, D), jnp.float32),
        pltpu.VMEM((16, 17), jnp.float32),
        pltpu.SemaphoreType.DMA,
        pltpu.SemaphoreType.DMA,
    ],
    compiler_params=pltpu.CompilerParams(needs_layout_passes=False),
)
def _sc_kernel(x_sotu_hbm, x_taxon_hbm, idx0_hbm, idx1_hbm, out_hbm,
               idx0_v, idx1_v, out_v, rows0a, rows1a, rows0b, rows1b,
               tr_v, sem_a, sem_b):
    wid = lax.axis_index("s") * 2 + lax.axis_index("c")
    base_w = wid * E_PER_W
    lane = lax.iota(jnp.int32, 16)

    pltpu.sync_copy(idx0_hbm.at[pl.ds(base_w, E_PER_W)], idx0_v)
    pltpu.sync_copy(idx1_hbm.at[pl.ds(base_w, E_PER_W)], idx1_v)

    def fire(it, r0, r1, sem):
        pltpu.async_copy(x_sotu_hbm.at[idx0_v.at[pl.ds(it * C, C)]], r0, sem)
        pltpu.async_copy(x_taxon_hbm.at[idx1_v.at[pl.ds(it * C, C)]], r1, sem)

    def drain(it, r0, r1, sem):
        pltpu.make_async_copy(
            x_sotu_hbm.at[idx0_v.at[pl.ds(it * C, C)]], r0, sem).wait()
        pltpu.make_async_copy(
            x_taxon_hbm.at[idx1_v.at[pl.ds(it * C, C)]], r1, sem).wait()

    def compute(it, r0, r1):
        def group_body(g, _):
            for e in range(16):
                row = g * 16 + e
                ps = []
                for k in range(8):
                    a = r0[row, pl.ds(16 * k, 16)]
                    b = r1[row, pl.ds(16 * k, 16)]
                    ps.append(a * b)
                s01 = ps[0] + ps[1]
                s23 = ps[2] + ps[3]
                s45 = ps[4] + ps[5]
                s67 = ps[6] + ps[7]
                tr_v[e, pl.ds(0, 16)] = (s01 + s23) + (s45 + s67)
            one = jnp.ones((16,), jnp.int32)
            col = jnp.zeros((16,), jnp.int32)
            acc = jnp.zeros((16,), jnp.float32)
            for c in range(16):
                acc = acc + plsc.load_gather(tr_v, [lane, col])
                col = col + one
            out_v[pl.ds(it * C + g * 16, 16)] = acc
            return 0

        lax.fori_loop(0, GROUPS, group_body, 0)

    fire(0, rows0a, rows1a, sem_a)

    def body(j, _):
        a = 2 * j
        fire(a + 1, rows0b, rows1b, sem_b)
        drain(a, rows0a, rows1a, sem_a)
        compute(a, rows0a, rows1a)
        fire(a + 2, rows0a, rows1a, sem_a)
        drain(a + 1, rows0b, rows1b, sem_b)
        compute(a + 1, rows0b, rows1b)
        return 0

    lax.fori_loop(0, (N_CHUNKS - 1) // 2, body, 0)
    drain(N_CHUNKS - 1, rows0a, rows1a, sem_a)
    compute(N_CHUNKS - 1, rows0a, rows1a)

    pltpu.sync_copy(out_v, out_hbm.at[pl.ds(base_w, E_PER_W)])


def kernel(x_sotu, x_taxon, edge_label_index):
    idx0 = edge_label_index[0]
    idx1 = edge_label_index[1]
    return _sc_kernel(x_sotu, x_taxon, idx0, idx1)
